# baseline (speedup 1.0000x reference)
import functools

import jax
import jax.numpy as jnp
from jax import lax
from jax.experimental import pallas as pl
from jax.experimental.pallas import tpu as pltpu
from jax.experimental.pallas import tpu_sc as plsc

_KAPPA_MAX = 0.7
_N0 = 10.0

_BN = 256


def _sc_gather(patch_ids, theta, n_eff):
    N = patch_ids.shape[0]
    info = plsc.get_sparse_core_info()
    NC, NS = info.num_cores, info.num_subcores
    NW = NC * NS
    bpw = N // NW
    mesh = plsc.VectorSubcoreMesh(core_axis_name="c", subcore_axis_name="s")

    @functools.partial(
        pl.kernel,
        mesh=mesh,
        out_type=[
            jax.ShapeDtypeStruct((N,), jnp.float32),
            jax.ShapeDtypeStruct((N,), jnp.float32),
        ],
        scratch_types=[
            pltpu.VMEM((bpw,), jnp.int32),
            pltpu.VMEM((bpw,), jnp.float32),
            pltpu.VMEM((bpw,), jnp.float32),
            pltpu.SemaphoreType.DMA,
            pltpu.SemaphoreType.DMA,
        ],
    )
    def gk(idx_hbm, theta_hbm, neff_hbm, th_out, ne_out, idx_v, th_v, ne_v, s1, s2):
        wid = lax.axis_index("s") * NC + lax.axis_index("c")
        base = wid * bpw
        pltpu.sync_copy(idx_hbm.at[pl.ds(base, bpw)], idx_v)
        c1 = pltpu.async_copy(theta_hbm.at[idx_v], th_v, s1)
        c2 = pltpu.async_copy(neff_hbm.at[idx_v], ne_v, s2)
        c1.wait()
        c2.wait()
        pltpu.sync_copy(th_v, th_out.at[pl.ds(base, bpw)])
        pltpu.sync_copy(ne_v, ne_out.at[pl.ds(base, bpw)])

    return gk(patch_ids, theta, n_eff)


def _mv_body(h_ref, w_ref, b_ref, raw_ref):
    w = w_ref[...]
    h = h_ref[...]
    raw_ref[...] = jnp.sum(h * w[None], axis=-1) + b_ref[0, 0]


def _comb_body(raw_ref, th_ref, ne_ref, lt_ref, probs_ref, ls_ref):
    raw = raw_ref[...]
    th = th_ref[...]
    ne = ne_ref[...]
    bl = jnp.log(th) - jnp.log(1.0 - th)
    kap = jnp.clip(_KAPPA_MAX * (_N0 / (ne + _N0)), 0.0, _KAPPA_MAX)
    ls = (1.0 - kap) * raw + kap * bl
    t = jnp.log(1.0 + jnp.exp(lt_ref[0, 0])) + 1e-4
    probs_ref[...] = 1.0 / (1.0 + jnp.exp(-ls / t))
    ls_ref[...] = ls


def kernel(H_t, patch_ids, theta, n_eff, W, b, log_temperature):
    S, N, D = H_t.shape
    pid = patch_ids.astype(jnp.int32)
    th_n, ne_n = _sc_gather(pid, theta, n_eff)

    BN = _BN
    raw = pl.pallas_call(
        _mv_body,
        grid=(N // BN,),
        in_specs=[
            pl.BlockSpec((S, BN, D), lambda i: (0, i, 0)),
            pl.BlockSpec((1, D), lambda i: (0, 0)),
            pl.BlockSpec(memory_space=pltpu.SMEM),
        ],
        out_specs=pl.BlockSpec((S, BN), lambda i: (0, i)),
        out_shape=jax.ShapeDtypeStruct((S, N), jnp.float32),
        compiler_params=pltpu.CompilerParams(
            dimension_semantics=("arbitrary",),
        ),
    )(H_t, W, b.reshape(1, 1))

    probs, ls = pl.pallas_call(
        _comb_body,
        in_specs=[
            pl.BlockSpec((S, N), lambda: (0, 0)),
            pl.BlockSpec((1, N), lambda: (0, 0)),
            pl.BlockSpec((1, N), lambda: (0, 0)),
            pl.BlockSpec(memory_space=pltpu.SMEM),
        ],
        out_specs=[
            pl.BlockSpec((S, N), lambda: (0, 0)),
            pl.BlockSpec((S, N), lambda: (0, 0)),
        ],
        out_shape=[
            jax.ShapeDtypeStruct((S, N), jnp.float32),
            jax.ShapeDtypeStruct((S, N), jnp.float32),
        ],
    )(raw, th_n.reshape(1, N), ne_n.reshape(1, N),
      log_temperature.astype(jnp.float32).reshape(1, 1))
    return probs, ls


# speedup vs baseline: 1.0044x; 1.0044x over previous
import functools

import jax
import jax.numpy as jnp
from jax import lax
from jax.experimental import pallas as pl
from jax.experimental.pallas import tpu as pltpu
from jax.experimental.pallas import tpu_sc as plsc

_KAPPA_MAX = 0.7
_N0 = 10.0

_BN = 1024


def _sc_gather(patch_ids, theta, n_eff):
    N = patch_ids.shape[0]
    info = plsc.get_sparse_core_info()
    NC, NS = info.num_cores, info.num_subcores
    NW = NC * NS
    bpw = N // NW
    mesh = plsc.VectorSubcoreMesh(core_axis_name="c", subcore_axis_name="s")

    @functools.partial(
        pl.kernel,
        mesh=mesh,
        out_type=[
            jax.ShapeDtypeStruct((N,), jnp.float32),
            jax.ShapeDtypeStruct((N,), jnp.float32),
        ],
        scratch_types=[
            pltpu.VMEM((bpw,), jnp.int32),
            pltpu.VMEM((bpw,), jnp.float32),
            pltpu.VMEM((bpw,), jnp.float32),
            pltpu.SemaphoreType.DMA,
            pltpu.SemaphoreType.DMA,
        ],
    )
    def gk(idx_hbm, theta_hbm, neff_hbm, th_out, ne_out, idx_v, th_v, ne_v, s1, s2):
        wid = lax.axis_index("s") * NC + lax.axis_index("c")
        base = wid * bpw
        pltpu.sync_copy(idx_hbm.at[pl.ds(base, bpw)], idx_v)
        c1 = pltpu.async_copy(theta_hbm.at[idx_v], th_v, s1)
        c2 = pltpu.async_copy(neff_hbm.at[idx_v], ne_v, s2)
        c1.wait()
        c2.wait()
        pltpu.sync_copy(th_v, th_out.at[pl.ds(base, bpw)])
        pltpu.sync_copy(ne_v, ne_out.at[pl.ds(base, bpw)])

    return gk(patch_ids, theta, n_eff)


def _mv_body(h_ref, w_ref, b_ref, raw_ref):
    w = w_ref[...]
    h = h_ref[...]
    raw_ref[...] = jnp.sum(h * w[None], axis=-1) + b_ref[0, 0]


def _comb_body(raw_ref, th_ref, ne_ref, lt_ref, probs_ref, ls_ref):
    raw = raw_ref[...]
    th = th_ref[...]
    ne = ne_ref[...]
    bl = jnp.log(th) - jnp.log(1.0 - th)
    kap = jnp.clip(_KAPPA_MAX * (_N0 / (ne + _N0)), 0.0, _KAPPA_MAX)
    ls = (1.0 - kap) * raw + kap * bl
    t = jnp.log(1.0 + jnp.exp(lt_ref[0, 0])) + 1e-4
    probs_ref[...] = 1.0 / (1.0 + jnp.exp(-ls / t))
    ls_ref[...] = ls


def kernel(H_t, patch_ids, theta, n_eff, W, b, log_temperature):
    S, N, D = H_t.shape
    pid = patch_ids.astype(jnp.int32)
    th_n, ne_n = _sc_gather(pid, theta, n_eff)

    BN = _BN
    raw = pl.pallas_call(
        _mv_body,
        grid=(N // BN,),
        in_specs=[
            pl.BlockSpec((S, BN, D), lambda i: (0, i, 0)),
            pl.BlockSpec((1, D), lambda i: (0, 0)),
            pl.BlockSpec(memory_space=pltpu.SMEM),
        ],
        out_specs=pl.BlockSpec((S, BN), lambda i: (0, i)),
        out_shape=jax.ShapeDtypeStruct((S, N), jnp.float32),
        compiler_params=pltpu.CompilerParams(
            dimension_semantics=("arbitrary",),
        ),
    )(H_t, W, b.reshape(1, 1))

    probs, ls = pl.pallas_call(
        _comb_body,
        in_specs=[
            pl.BlockSpec((S, N), lambda: (0, 0)),
            pl.BlockSpec((1, N), lambda: (0, 0)),
            pl.BlockSpec((1, N), lambda: (0, 0)),
            pl.BlockSpec(memory_space=pltpu.SMEM),
        ],
        out_specs=[
            pl.BlockSpec((S, N), lambda: (0, 0)),
            pl.BlockSpec((S, N), lambda: (0, 0)),
        ],
        out_shape=[
            jax.ShapeDtypeStruct((S, N), jnp.float32),
            jax.ShapeDtypeStruct((S, N), jnp.float32),
        ],
    )(raw, th_n.reshape(1, N), ne_n.reshape(1, N),
      log_temperature.astype(jnp.float32).reshape(1, 1))
    return probs, ls


# speedup vs baseline: 1.0264x; 1.0219x over previous
import functools

import jax
import jax.numpy as jnp
from jax import lax
from jax.experimental import pallas as pl
from jax.experimental.pallas import tpu as pltpu
from jax.experimental.pallas import tpu_sc as plsc

_KAPPA_MAX = 0.7
_N0 = 10.0

_BN = 512


def _sc_gather(patch_ids, theta, n_eff):
    N = patch_ids.shape[0]
    info = plsc.get_sparse_core_info()
    NC, NS = info.num_cores, info.num_subcores
    NW = NC * NS
    bpw = N // NW
    mesh = plsc.VectorSubcoreMesh(core_axis_name="c", subcore_axis_name="s")

    @functools.partial(
        pl.kernel,
        mesh=mesh,
        out_type=[
            jax.ShapeDtypeStruct((N,), jnp.float32),
            jax.ShapeDtypeStruct((N,), jnp.float32),
        ],
        scratch_types=[
            pltpu.VMEM((bpw,), jnp.int32),
            pltpu.VMEM((bpw,), jnp.float32),
            pltpu.VMEM((bpw,), jnp.float32),
            pltpu.SemaphoreType.DMA,
            pltpu.SemaphoreType.DMA,
        ],
    )
    def gk(idx_hbm, theta_hbm, neff_hbm, th_out, ne_out, idx_v, th_v, ne_v, s1, s2):
        wid = lax.axis_index("s") * NC + lax.axis_index("c")
        base = wid * bpw
        pltpu.sync_copy(idx_hbm.at[pl.ds(base, bpw)], idx_v)
        c1 = pltpu.async_copy(theta_hbm.at[idx_v], th_v, s1)
        c2 = pltpu.async_copy(neff_hbm.at[idx_v], ne_v, s2)
        c1.wait()
        c2.wait()
        pltpu.sync_copy(th_v, th_out.at[pl.ds(base, bpw)])
        pltpu.sync_copy(ne_v, ne_out.at[pl.ds(base, bpw)])

    return gk(patch_ids, theta, n_eff)


def _mv_body(h0_ref, h1_ref, w_ref, b_ref, raw_ref):
    w = w_ref[...]
    b = b_ref[0, 0]
    raw_ref[:2] = jnp.sum(h0_ref[...] * w[None], axis=-1) + b
    raw_ref[2:] = jnp.sum(h1_ref[...] * w[None], axis=-1) + b


def _comb_body(raw_ref, th_ref, ne_ref, lt_ref, probs_ref, ls_ref):
    raw = raw_ref[...]
    th = th_ref[...]
    ne = ne_ref[...]
    bl = jnp.log(th) - jnp.log(1.0 - th)
    kap = jnp.clip(_KAPPA_MAX * (_N0 / (ne + _N0)), 0.0, _KAPPA_MAX)
    ls = (1.0 - kap) * raw + kap * bl
    t = jnp.log(1.0 + jnp.exp(lt_ref[0, 0])) + 1e-4
    probs_ref[...] = 1.0 / (1.0 + jnp.exp(-ls / t))
    ls_ref[...] = ls


def kernel(H_t, patch_ids, theta, n_eff, W, b, log_temperature):
    S, N, D = H_t.shape
    pid = patch_ids.astype(jnp.int32)
    th_n, ne_n = _sc_gather(pid, theta, n_eff)

    BN = _BN
    raw = pl.pallas_call(
        _mv_body,
        grid=(N // BN,),
        in_specs=[
            pl.BlockSpec((2, BN, D), lambda i: (0, i, 0)),
            pl.BlockSpec((2, BN, D), lambda i: (1, i, 0)),
            pl.BlockSpec((1, D), lambda i: (0, 0)),
            pl.BlockSpec(memory_space=pltpu.SMEM),
        ],
        out_specs=pl.BlockSpec((S, BN), lambda i: (0, i)),
        out_shape=jax.ShapeDtypeStruct((S, N), jnp.float32),
        compiler_params=pltpu.CompilerParams(
            dimension_semantics=("arbitrary",),
        ),
    )(H_t, H_t, W, b.reshape(1, 1))

    probs, ls = pl.pallas_call(
        _comb_body,
        in_specs=[
            pl.BlockSpec((S, N), lambda: (0, 0)),
            pl.BlockSpec((1, N), lambda: (0, 0)),
            pl.BlockSpec((1, N), lambda: (0, 0)),
            pl.BlockSpec(memory_space=pltpu.SMEM),
        ],
        out_specs=[
            pl.BlockSpec((S, N), lambda: (0, 0)),
            pl.BlockSpec((S, N), lambda: (0, 0)),
        ],
        out_shape=[
            jax.ShapeDtypeStruct((S, N), jnp.float32),
            jax.ShapeDtypeStruct((S, N), jnp.float32),
        ],
    )(raw, th_n.reshape(1, N), ne_n.reshape(1, N),
      log_temperature.astype(jnp.float32).reshape(1, 1))
    return probs, ls


# speedup vs baseline: 1.0320x; 1.0055x over previous
import functools

import jax
import jax.numpy as jnp
from jax import lax
from jax.experimental import pallas as pl
from jax.experimental.pallas import tpu as pltpu
from jax.experimental.pallas import tpu_sc as plsc

_KAPPA_MAX = 0.7
_N0 = 10.0

_BN = 512


def _sc_gather(patch_ids, theta, n_eff):
    N = patch_ids.shape[0]
    info = plsc.get_sparse_core_info()
    NC, NS = info.num_cores, info.num_subcores
    NW = NC * NS
    bpw = N // NW
    mesh = plsc.VectorSubcoreMesh(core_axis_name="c", subcore_axis_name="s")

    @functools.partial(
        pl.kernel,
        mesh=mesh,
        out_type=[
            jax.ShapeDtypeStruct((N,), jnp.float32),
            jax.ShapeDtypeStruct((N,), jnp.float32),
        ],
        scratch_types=[
            pltpu.VMEM((bpw,), jnp.int32),
            pltpu.VMEM((bpw,), jnp.float32),
            pltpu.VMEM((bpw,), jnp.float32),
            pltpu.SemaphoreType.DMA,
            pltpu.SemaphoreType.DMA,
        ],
    )
    def gk(idx_hbm, theta_hbm, neff_hbm, th_out, ne_out, idx_v, th_v, ne_v, s1, s2):
        wid = lax.axis_index("s") * NC + lax.axis_index("c")
        base = wid * bpw
        pltpu.sync_copy(idx_hbm.at[pl.ds(base, bpw)], idx_v)
        c1 = pltpu.async_copy(theta_hbm.at[idx_v], th_v, s1)
        c2 = pltpu.async_copy(neff_hbm.at[idx_v], ne_v, s2)
        c1.wait()
        c2.wait()
        pltpu.sync_copy(th_v, th_out.at[pl.ds(base, bpw)])
        pltpu.sync_copy(ne_v, ne_out.at[pl.ds(base, bpw)])

    return gk(patch_ids, theta, n_eff)


def _mv_body(h_ref, w_ref, b_ref, raw_ref):
    w = w_ref[...]
    h = h_ref[...]
    raw_ref[...] = jnp.sum(h * w[None], axis=-1) + b_ref[0, 0]


def _comb_body(raw_ref, th_ref, ne_ref, lt_ref, probs_ref, ls_ref):
    raw = raw_ref[...]
    th = th_ref[...]
    ne = ne_ref[...]
    bl = jnp.log(th) - jnp.log(1.0 - th)
    kap = jnp.clip(_KAPPA_MAX * (_N0 / (ne + _N0)), 0.0, _KAPPA_MAX)
    ls = (1.0 - kap) * raw + kap * bl
    t = jnp.log(1.0 + jnp.exp(lt_ref[0, 0])) + 1e-4
    probs_ref[...] = 1.0 / (1.0 + jnp.exp(-ls / t))
    ls_ref[...] = ls


def kernel(H_t, patch_ids, theta, n_eff, W, b, log_temperature):
    S, N, D = H_t.shape
    pid = patch_ids.astype(jnp.int32)
    th_n, ne_n = _sc_gather(pid, theta, n_eff)

    BN = _BN
    raw = pl.pallas_call(
        _mv_body,
        grid=(N // BN,),
        in_specs=[
            pl.BlockSpec((S, BN, D), lambda i: (0, i, 0)),
            pl.BlockSpec((1, D), lambda i: (0, 0)),
            pl.BlockSpec(memory_space=pltpu.SMEM),
        ],
        out_specs=pl.BlockSpec((S, BN), lambda i: (0, i)),
        out_shape=jax.ShapeDtypeStruct((S, N), jnp.float32),
        compiler_params=pltpu.CompilerParams(
            dimension_semantics=("arbitrary",),
        ),
    )(H_t, W, b.reshape(1, 1))

    probs, ls = pl.pallas_call(
        _comb_body,
        in_specs=[
            pl.BlockSpec((S, N), lambda: (0, 0)),
            pl.BlockSpec((1, N), lambda: (0, 0)),
            pl.BlockSpec((1, N), lambda: (0, 0)),
            pl.BlockSpec(memory_space=pltpu.SMEM),
        ],
        out_specs=[
            pl.BlockSpec((S, N), lambda: (0, 0)),
            pl.BlockSpec((S, N), lambda: (0, 0)),
        ],
        out_shape=[
            jax.ShapeDtypeStruct((S, N), jnp.float32),
            jax.ShapeDtypeStruct((S, N), jnp.float32),
        ],
    )(raw, th_n.reshape(1, N), ne_n.reshape(1, N),
      log_temperature.astype(jnp.float32).reshape(1, 1))
    return probs, ls
